# baseline (speedup 1.0000x reference)
import functools

import jax
import jax.numpy as jnp
from jax import lax
from jax.experimental import pallas as pl
from jax.experimental.pallas import tpu as pltpu
from jax.experimental.pallas import tpu_sc as plsc

_info = plsc.get_sparse_core_info()
_NC, _NS, _NL = _info.num_cores, _info.num_subcores, _info.num_lanes
_NW = _NC * _NS

_CHUNK = 256
_NBUF = 4


def _gather_rows(table, idx, idx2_shape):
    V, D = table.shape
    B = idx.shape[0]
    b_per_w = B // _NW
    n_chunks = b_per_w // _CHUNK
    assert B % (_NW * _CHUNK) == 0 and n_chunks % _NBUF == 0
    mesh = plsc.VectorSubcoreMesh(core_axis_name="c", subcore_axis_name="s")

    T, Bt = idx2_shape
    @functools.partial(
        pl.kernel,
        mesh=mesh,
        compiler_params=pltpu.CompilerParams(use_tc_tiling_on_sc=False),
        out_type=jax.ShapeDtypeStruct((T, Bt, D), jnp.float32),
        scratch_types=(
            [pltpu.VMEM((b_per_w,), jnp.int32)]
            + [pltpu.VMEM((_CHUNK, D), jnp.float32) for _ in range(_NBUF)]
            + [pltpu.SemaphoreType.DMA for _ in range(2 * _NBUF)]
        ),
    )
    def k(table_hbm, idx_hbm, out3_hbm, idx_all, r0, r1, r2, r3,
          g0, g1, g2, g3, o0, o1, o2, o3):
        rows = (r0, r1, r2, r3)
        gsem = (g0, g1, g2, g3)
        osem = (o0, o1, o2, o3)
        wid = lax.axis_index("s") * _NC + lax.axis_index("c")
        base_w = wid * b_per_w
        pltpu.sync_copy(idx_hbm.at[pl.ds(base_w, b_per_w)], idx_all)

        def issue_gather(c, b):
            pltpu.async_copy(
                table_hbm.at[idx_all.at[pl.ds(c * _CHUNK, _CHUNK)]],
                rows[b], gsem[b],
            )

        for c in range(2):
            issue_gather(c, c)

        def quad_body(t, carry):
            for j in range(_NBUF):
                c = _NBUF * t + j
                b2 = (j + 2) % _NBUF
                pltpu.make_async_copy(
                    table_hbm.at[idx_all.at[pl.ds(0, _CHUNK)]],
                    rows[j], gsem[j],
                ).wait()
                pos = base_w + c * _CHUNK
                pltpu.async_copy(
                    rows[j],
                    out3_hbm.at[pos // Bt, pl.ds(pos % Bt, _CHUNK)],
                    osem[j],
                )

                @pl.when(c + 2 < n_chunks)
                def _():
                    @pl.when(c >= 2)
                    def _():
                        pltpu.make_async_copy(
                            rows[b2], out3_hbm.at[0, pl.ds(0, _CHUNK)], osem[b2]
                        ).wait()

                    issue_gather(c + 2, b2)

            return carry

        lax.fori_loop(0, n_chunks // _NBUF, quad_body, 0)
        for j in range(_NBUF):
            pltpu.make_async_copy(
                rows[j], out3_hbm.at[0, pl.ds(0, _CHUNK)], osem[j]
            ).wait()

    return k(table, idx)


def kernel(input, W):
    Bv, T = input.shape
    idx = input.T.reshape(-1).astype(jnp.int32)
    out3 = _gather_rows(W, idx, (T, Bv))
    return out3.transpose(1, 0, 2)
